# Initial kernel scaffold; baseline (speedup 1.0000x reference)
import jax
import jax.numpy as jnp
from jax.experimental import pallas as pl


def kernel(ids, adj_tail, adj, feats, W_x1, b_x1, W_n1, b_n1, W_x2, b_x2, W_n2, b_n2, W_mu, b_mu, W_lv, b_lv, W_rec, b_rec):
    raise NotImplementedError("write your pallas kernel here")


# speedup vs baseline: 4.3066x; 4.3066x over previous
import functools

import jax
import jax.numpy as jnp
import numpy as np
from jax import lax
from jax.experimental import pallas as pl
from jax.experimental.pallas import tpu as pltpu
from jax.experimental.pallas import tpu_sc as plsc

N = 10000
D = 256
DEG = 16
B = 1024
NS = 10
H = 128
HID = 2 * H

NW = 32
RB = B // NW
L1 = RB * NS
L2 = L1 * NS


def _sample_consts():
    key = jax.random.key(42)
    cols = {}
    for s in range(2):
        ks = jax.random.fold_in(key, s)
        for i in range(2):
            k = jax.random.fold_in(ks, i)
            cols[(s, i)] = jax.random.permutation(k, DEG)[:NS].astype(jnp.int32)
    eps = jax.random.normal(jax.random.fold_in(key, 2), (B, HID), jnp.float32)
    return cols, eps


def _pre_body(feats_ref, w_ref, b_ref, pq_ref, p_ref):
    acc = jnp.dot(feats_ref[...], w_ref[...],
                  preferred_element_type=jnp.float32) + b_ref[...]
    pq_ref[...] = acc
    p_ref[...] = acc[:, H:]


_pre = pl.pallas_call(
    _pre_body,
    grid=(10,),
    in_specs=[pl.BlockSpec((N // 10, D), lambda i: (i, 0)),
              pl.BlockSpec((D, HID), lambda i: (0, 0)),
              pl.BlockSpec((1, HID), lambda i: (0, 0))],
    out_specs=[pl.BlockSpec((N // 10, HID), lambda i: (i, 0)),
               pl.BlockSpec((N // 10, H), lambda i: (i, 0))],
    out_shape=[jax.ShapeDtypeStruct((N, HID), jnp.float32),
               jax.ShapeDtypeStruct((N, H), jnp.float32)],
)


def _sc_body(ids_hbm, adjt_hbm, adjx_hbm, r10_hbm, r100_hbm,
             cAt_hbm, cBt_hbm, cCt_hbm, cAx_hbm, cBx_hbm, cCx_hbm,
             pq_hbm, pn_hbm,
             h0t_hbm, h1mt_hbm, h0x_hbm, h1mx_hbm,
             ids_v, r10_v, r100_v, idr10_v, idr100_v,
             cA_v, cB_v, cC_v, f1_v, cur1_v, fr_v, cur1r_v, cur2_v,
             pq0, pq1, gbuf, h0_acc, h1m_acc, sem, sem2):
    wid = lax.axis_index("s") * 2 + lax.axis_index("c")
    base = wid * RB
    tenth = jnp.float32(0.1)

    pltpu.sync_copy(ids_hbm.at[pl.ds(base, RB)], ids_v)
    pltpu.sync_copy(r10_hbm.at[pl.ds(base * NS, L1)], r10_v)
    pltpu.sync_copy(r100_hbm.at[pl.ds(base * NS * NS, L2)], r100_v)
    pltpu.async_copy(ids_hbm.at[r10_v], idr10_v, sem).wait()
    pltpu.async_copy(ids_hbm.at[r100_v], idr100_v, sem).wait()
    pltpu.async_copy(pq_hbm.at[ids_v], pq0, sem).wait()

    def _fcomp(dst, srcA, srcB, nslices):
        def body(t, _):
            sl = pl.ds(16 * t, 16)
            dst[sl] = (srcA[sl] << 4) + srcB[sl]
            return 0
        lax.fori_loop(0, nslices, body, 0)

    for adjf_hbm, cA_hbm, cB_hbm, cC_hbm, h0_hbm, h1m_hbm in (
            (adjt_hbm, cAt_hbm, cBt_hbm, cCt_hbm, h0t_hbm, h1mt_hbm),
            (adjx_hbm, cAx_hbm, cBx_hbm, cCx_hbm, h0x_hbm, h1mx_hbm)):
        pltpu.sync_copy(cA_hbm.at[pl.ds(base * NS, L1)], cA_v)
        pltpu.sync_copy(cB_hbm.at[pl.ds(base * NS * NS, L2)], cB_v)
        pltpu.sync_copy(cC_hbm.at[pl.ds(base * NS * NS, L2)], cC_v)

        _fcomp(f1_v, idr10_v, cA_v, L1 // 16)
        pltpu.async_copy(adjf_hbm.at[f1_v], cur1_v, sem).wait()
        _fcomp(fr_v, idr100_v, cB_v, L2 // 16)
        pltpu.async_copy(adjf_hbm.at[fr_v], cur1r_v, sem).wait()
        _fcomp(fr_v, cur1r_v, cC_v, L2 // 16)
        pltpu.async_copy(adjf_hbm.at[fr_v], cur2_v, sem).wait()

        def cc_body(cc, _):
            pltpu.async_copy(pq_hbm.at[cur1_v.at[pl.ds(80 * cc, 80)]],
                             pq1, sem).wait()

            def pp_body(pp, _):
                pltpu.async_copy(
                    pn_hbm.at[cur2_v.at[pl.ds(800 * cc + 200 * pp, 200)]],
                    gbuf, sem2).wait()
                for q in range(2):
                    g0 = 8 * cc + 2 * pp + q
                    rowb = NS * (2 * pp + q)
                    gb = 100 * q
                    for u in range(8):
                        sl = pl.ds(16 * u, 16)
                        slh = pl.ds(128 + 16 * u, 16)
                        xacc = jnp.maximum(pq1[rowb, sl], 0.0)
                        nacc0 = pq1[rowb, slh]
                        sv = gbuf[gb, sl]
                        for rr in range(1, NS):
                            sv = sv + gbuf[gb + rr, sl]
                        nacc = jnp.maximum(sv * tenth, 0.0)
                        for jl in range(1, NS):
                            row = rowb + jl
                            xacc = xacc + jnp.maximum(pq1[row, sl], 0.0)
                            nacc0 = nacc0 + pq1[row, slh]
                            sv = gbuf[gb + NS * jl, sl]
                            for rr in range(1, NS):
                                sv = sv + gbuf[gb + NS * jl + rr, sl]
                            nacc = nacc + jnp.maximum(sv * tenth, 0.0)
                        h1m_acc[g0, sl] = xacc * tenth
                        h1m_acc[g0, slh] = nacc * tenth
                        h0_acc[g0, slh] = nacc0 * tenth
                return 0
            lax.fori_loop(0, 4, pp_body, 0)
            return 0
        lax.fori_loop(0, RB // 8, cc_body, 0)

        def fin(i, _):
            for u in range(8):
                sl = pl.ds(16 * u, 16)
                slh = pl.ds(128 + 16 * u, 16)
                h0_acc[i, sl] = jnp.maximum(pq0[i, sl], 0.0)
                h0_acc[i, slh] = jnp.maximum(h0_acc[i, slh], 0.0)
            return 0
        lax.fori_loop(0, RB, fin, 0)

        pltpu.sync_copy(h0_acc, h0_hbm.at[pl.ds(base, RB)])
        pltpu.sync_copy(h1m_acc, h1m_hbm.at[pl.ds(base, RB)])


_sc = functools.partial(
    pl.kernel,
    mesh=plsc.VectorSubcoreMesh(core_axis_name="c", subcore_axis_name="s"),
    out_type=[jax.ShapeDtypeStruct((B, HID), jnp.float32)] * 4,
    scratch_types=[
        pltpu.VMEM((RB,), jnp.int32),
        pltpu.VMEM((L1,), jnp.int32),
        pltpu.VMEM((L2,), jnp.int32),
        pltpu.VMEM((L1,), jnp.int32),
        pltpu.VMEM((L2,), jnp.int32),
        pltpu.VMEM((L1,), jnp.int32),
        pltpu.VMEM((L2,), jnp.int32),
        pltpu.VMEM((L2,), jnp.int32),
        pltpu.VMEM((L1,), jnp.int32),
        pltpu.VMEM((L1,), jnp.int32),
        pltpu.VMEM((L2,), jnp.int32),
        pltpu.VMEM((L2,), jnp.int32),
        pltpu.VMEM((L2,), jnp.int32),
        pltpu.VMEM((RB, HID), jnp.float32),
        pltpu.VMEM((80, HID), jnp.float32),
        pltpu.VMEM((200, H), jnp.float32),
        pltpu.VMEM((RB, HID), jnp.float32),
        pltpu.VMEM((RB, HID), jnp.float32),
        pltpu.SemaphoreType.DMA,
        pltpu.SemaphoreType.DMA,
    ],
)(_sc_body)


def _post_body(h0t, h1mt, h0x, h1mx, wx2, bx2, wn2, bn2,
               wmu, bmu, wlv, blv, wrec, brec, eps,
               x_out, rec_out, mu_out, lv_out, z_out):
    def dot(a, b):
        return jnp.dot(a, b, preferred_element_type=jnp.float32)

    st = jnp.concatenate([dot(h0t[...], wx2[...]) + bx2[...],
                          dot(h1mt[...], wn2[...]) + bn2[...]], axis=1)
    sx = jnp.concatenate([dot(h0x[...], wx2[...]) + bx2[...],
                          dot(h1mx[...], wn2[...]) + bn2[...]], axis=1)
    x_out[...] = sx
    mu = dot(st, wmu[...]) + bmu[...]
    lv = dot(st, wlv[...]) + blv[...]
    mu_out[...] = mu
    lv_out[...] = lv
    std = jnp.exp(0.5 * lv)
    z = mu + eps[...] * std
    z_out[...] = z
    r = dot(z, wrec[...]) + brec[...]
    rec_out[...] = 1.0 / (1.0 + jnp.exp(-r))


_post = pl.pallas_call(
    _post_body,
    out_shape=[jax.ShapeDtypeStruct((B, HID), jnp.float32)] * 5,
)


def kernel(ids, adj_tail, adj, feats, W_x1, b_x1, W_n1, b_n1,
           W_x2, b_x2, W_n2, b_n2, W_mu, b_mu, W_lv, b_lv, W_rec, b_rec):
    wcat = jnp.concatenate([W_x1, W_n1], axis=1)
    bcat = jnp.concatenate([b_x1, b_n1]).reshape(1, HID)
    pq, pn = _pre(feats, wcat, bcat)
    cols, eps = _sample_consts()
    rep10 = jnp.asarray(np.arange(B * NS, dtype=np.int32) // NS)
    rep100 = jnp.asarray(np.arange(B * NS * NS, dtype=np.int32) // (NS * NS))

    def expand(s):
        c1, c2 = cols[(s, 0)], cols[(s, 1)]
        cA = jnp.tile(c1, B)
        cB = jnp.tile(jnp.repeat(c1, NS), B)
        cC = jnp.tile(c2, B * NS)
        return cA, cB, cC

    cAt, cBt, cCt = expand(0)
    cAx, cBx, cCx = expand(1)
    h0t, h1mt, h0x, h1mx = _sc(
        ids.astype(jnp.int32), adj_tail.reshape(-1), adj.reshape(-1),
        rep10, rep100, cAt, cBt, cCt, cAx, cBx, cCx, pq, pn)
    x, rec, mu, lv, z = _post(
        h0t, h1mt, h0x, h1mx,
        W_x2, b_x2.reshape(1, H), W_n2, b_n2.reshape(1, H),
        W_mu, b_mu.reshape(1, HID), W_lv, b_lv.reshape(1, HID),
        W_rec, b_rec.reshape(1, HID), eps)
    return (x, rec, mu, lv, z)


# speedup vs baseline: 6.2117x; 1.4424x over previous
import functools

import jax
import jax.numpy as jnp
import numpy as np
from jax import lax
from jax.experimental import pallas as pl
from jax.experimental.pallas import tpu as pltpu
from jax.experimental.pallas import tpu_sc as plsc

N = 10000
D = 256
DEG = 16
B = 1024
NS = 10
H = 128
HID = 2 * H

NW = 32
RB = B // NW
L1 = RB * NS
L2 = L1 * NS


def _sample_consts():
    key = jax.random.key(42)
    cols = {}
    for s in range(2):
        ks = jax.random.fold_in(key, s)
        for i in range(2):
            k = jax.random.fold_in(ks, i)
            cols[(s, i)] = jax.random.permutation(k, DEG)[:NS].astype(jnp.int32)
    eps = jax.random.normal(jax.random.fold_in(key, 2), (B, HID), jnp.float32)
    return cols, eps


def _pre_body(feats_ref, w_ref, b_ref, pq_ref, p_ref):
    acc = jnp.dot(feats_ref[...], w_ref[...],
                  preferred_element_type=jnp.float32) + b_ref[...]
    pq_ref[...] = acc
    p_ref[...] = acc[:, H:]


_pre = pl.pallas_call(
    _pre_body,
    grid=(10,),
    in_specs=[pl.BlockSpec((N // 10, D), lambda i: (i, 0)),
              pl.BlockSpec((D, HID), lambda i: (0, 0)),
              pl.BlockSpec((1, HID), lambda i: (0, 0))],
    out_specs=[pl.BlockSpec((N // 10, HID), lambda i: (i, 0)),
               pl.BlockSpec((N // 10, H), lambda i: (i, 0))],
    out_shape=[jax.ShapeDtypeStruct((N, HID), jnp.float32),
               jax.ShapeDtypeStruct((N, H), jnp.float32)],
)


def _sc_body(ids_hbm, adjt_hbm, adjx_hbm, r10_hbm, r100_hbm,
             cAt_hbm, cBt_hbm, cCt_hbm, cAx_hbm, cBx_hbm, cCx_hbm,
             pq_hbm, pn_hbm,
             h0t_hbm, h1mt_hbm, h0x_hbm, h1mx_hbm,
             ids_v, r10_v, r100_v, idr10_v, idr100_v,
             cAt_v, cAx_v, cB_v, cC_v, f1t_v, f1x_v, frt_v, frx_v,
             cur1r_v, cur1t_v, cur1x_v, cur2t_v, cur2x_v,
             pq0, pq1, gbuf0, gbuf1, h0_acc, h1m_acc,
             sA, sB, sC, sD, sE, sF, sG, sH, sP, sg0, sg1):
    wid = lax.axis_index("s") * 2 + lax.axis_index("c")
    base = wid * RB
    tenth = jnp.float32(0.1)

    def _fcomp(dst, srcA, srcB, nslices):
        def body(t, _):
            sl = pl.ds(16 * t, 16)
            dst[sl] = (srcA[sl] << 4) + srcB[sl]
            return 0
        lax.fori_loop(0, nslices, body, 0)

    a_ids = pltpu.async_copy(ids_hbm.at[pl.ds(base, RB)], ids_v, sA)
    a_r10 = pltpu.async_copy(r10_hbm.at[pl.ds(base * NS, L1)], r10_v, sB)
    a_r100 = pltpu.async_copy(r100_hbm.at[pl.ds(base * NS * NS, L2)],
                              r100_v, sC)
    a_cAt = pltpu.async_copy(cAt_hbm.at[pl.ds(base * NS, L1)], cAt_v, sD)
    a_cAx = pltpu.async_copy(cAx_hbm.at[pl.ds(base * NS, L1)], cAx_v, sE)
    a_cB = pltpu.async_copy(cBt_hbm.at[pl.ds(base * NS * NS, L2)], cB_v, sF)
    a_cC = pltpu.async_copy(cCt_hbm.at[pl.ds(base * NS * NS, L2)], cC_v, sG)
    a_ids.wait()
    a_pq0 = pltpu.async_copy(pq_hbm.at[ids_v], pq0, sH)
    a_r10.wait()
    a_idr10 = pltpu.async_copy(ids_hbm.at[r10_v], idr10_v, sA)
    a_r100.wait()
    a_idr100 = pltpu.async_copy(ids_hbm.at[r100_v], idr100_v, sB)
    a_idr10.wait()
    a_cAt.wait()
    _fcomp(f1t_v, idr10_v, cAt_v, L1 // 16)
    a_cur1t = pltpu.async_copy(adjt_hbm.at[f1t_v], cur1t_v, sA)
    a_cAx.wait()
    _fcomp(f1x_v, idr10_v, cAx_v, L1 // 16)
    a_cur1x = pltpu.async_copy(adjx_hbm.at[f1x_v], cur1x_v, sE)
    a_idr100.wait()
    a_cB.wait()
    _fcomp(frt_v, idr100_v, cB_v, L2 // 16)
    a_c1rt = pltpu.async_copy(adjt_hbm.at[frt_v], cur1r_v, sB)
    a_cBx = pltpu.async_copy(cBx_hbm.at[pl.ds(base * NS * NS, L2)], cB_v, sF)
    a_c1rt.wait()
    a_cC.wait()
    _fcomp(frt_v, cur1r_v, cC_v, L2 // 16)
    a_cur2t = pltpu.async_copy(adjt_hbm.at[frt_v], cur2t_v, sB)
    a_cCx = pltpu.async_copy(cCx_hbm.at[pl.ds(base * NS * NS, L2)], cC_v, sG)
    a_cBx.wait()
    _fcomp(frx_v, idr100_v, cB_v, L2 // 16)
    a_c1rx = pltpu.async_copy(adjx_hbm.at[frx_v], cur1r_v, sC)
    a_c1rx.wait()
    a_cCx.wait()
    _fcomp(frx_v, cur1r_v, cC_v, L2 // 16)
    a_cur2x = pltpu.async_copy(adjx_hbm.at[frx_v], cur2x_v, sC)

    gbufs = (gbuf0, gbuf1)
    sgs = (sg0, sg1)

    for bi, (cur1_v, cur2_v, a_cur1, a_cur2, h0_hbm, h1m_hbm) in enumerate((
            (cur1t_v, cur2t_v, a_cur1t, a_cur2t, h0t_hbm, h1mt_hbm),
            (cur1x_v, cur2x_v, a_cur1x, a_cur2x, h0x_hbm, h1mx_hbm))):
        a_cur1.wait()
        a_cur2.wait()
        pltpu.async_copy(pn_hbm.at[cur2_v.at[pl.ds(0, 200)]], gbuf0, sg0)
        pltpu.async_copy(pn_hbm.at[cur2_v.at[pl.ds(200, 200)]], gbuf1, sg1)

        def oo_body(oo, _):
            for b in range(2):
                c = 2 * oo + b

                if b == 0:
                    @pl.when((c & 3) == 0)
                    def _reload():
                        pltpu.async_copy(
                            pq_hbm.at[cur1_v.at[pl.ds(80 * (oo >> 1), 80)]],
                            pq1, sP).wait()

                pltpu.make_async_copy(
                    pn_hbm.at[cur2_v.at[pl.ds(0, 200)]], gbufs[b],
                    sgs[b]).wait()

                for q in range(2):
                    g0 = 2 * c + q
                    rowb = (g0 & 7) * NS
                    gb = 100 * q
                    gbuf = gbufs[b]

                    def u_body(u, _, g0=g0, rowb=rowb, gb=gb, gbuf=gbuf):
                        sl = pl.ds(16 * u, 16)
                        slh = pl.ds(128 + 16 * u, 16)
                        xacc = jnp.maximum(pq1[rowb, sl], 0.0)
                        nacc0 = pq1[rowb, slh]
                        sv = gbuf[gb, sl]
                        for rr in range(1, NS):
                            sv = sv + gbuf[gb + rr, sl]
                        nacc = jnp.maximum(sv * tenth, 0.0)
                        for jl in range(1, NS):
                            row = rowb + jl
                            xacc = xacc + jnp.maximum(pq1[row, sl], 0.0)
                            nacc0 = nacc0 + pq1[row, slh]
                            sv = gbuf[gb + NS * jl, sl]
                            for rr in range(1, NS):
                                sv = sv + gbuf[gb + NS * jl + rr, sl]
                            nacc = nacc + jnp.maximum(sv * tenth, 0.0)
                        h1m_acc[g0, sl] = xacc * tenth
                        h1m_acc[g0, slh] = nacc * tenth
                        h0_acc[g0, slh] = nacc0 * tenth
                        return 0
                    lax.fori_loop(0, 8, u_body, 0)

                @pl.when(oo < 7)
                def _next():
                    pltpu.async_copy(
                        pn_hbm.at[cur2_v.at[pl.ds(200 * (c + 2), 200)]],
                        gbufs[b], sgs[b])
            return 0
        lax.fori_loop(0, 8, oo_body, 0)

        if bi == 0:
            a_pq0.wait()

        def fin(i, _):
            for u in range(8):
                sl = pl.ds(16 * u, 16)
                slh = pl.ds(128 + 16 * u, 16)
                h0_acc[i, sl] = jnp.maximum(pq0[i, sl], 0.0)
                h0_acc[i, slh] = jnp.maximum(h0_acc[i, slh], 0.0)
            return 0
        lax.fori_loop(0, RB, fin, 0)

        pltpu.sync_copy(h0_acc, h0_hbm.at[pl.ds(base, RB)])
        pltpu.sync_copy(h1m_acc, h1m_hbm.at[pl.ds(base, RB)])


_sc = functools.partial(
    pl.kernel,
    mesh=plsc.VectorSubcoreMesh(core_axis_name="c", subcore_axis_name="s"),
    out_type=[jax.ShapeDtypeStruct((B, HID), jnp.float32)] * 4,
    scratch_types=[
        pltpu.VMEM((RB,), jnp.int32),
        pltpu.VMEM((L1,), jnp.int32),
        pltpu.VMEM((L2,), jnp.int32),
        pltpu.VMEM((L1,), jnp.int32),
        pltpu.VMEM((L2,), jnp.int32),
        pltpu.VMEM((L1,), jnp.int32),
        pltpu.VMEM((L1,), jnp.int32),
        pltpu.VMEM((L2,), jnp.int32),
        pltpu.VMEM((L2,), jnp.int32),
        pltpu.VMEM((L1,), jnp.int32),
        pltpu.VMEM((L1,), jnp.int32),
        pltpu.VMEM((L2,), jnp.int32),
        pltpu.VMEM((L2,), jnp.int32),
        pltpu.VMEM((L2,), jnp.int32),
        pltpu.VMEM((L1,), jnp.int32),
        pltpu.VMEM((L1,), jnp.int32),
        pltpu.VMEM((L2,), jnp.int32),
        pltpu.VMEM((L2,), jnp.int32),
        pltpu.VMEM((RB, HID), jnp.float32),
        pltpu.VMEM((80, HID), jnp.float32),
        pltpu.VMEM((200, H), jnp.float32),
        pltpu.VMEM((200, H), jnp.float32),
        pltpu.VMEM((RB, HID), jnp.float32),
        pltpu.VMEM((RB, HID), jnp.float32),
    ] + [pltpu.SemaphoreType.DMA] * 11,
)(_sc_body)


def _post_body(h0t, h1mt, h0x, h1mx, wx2, bx2, wn2, bn2,
               wmu, bmu, wlv, blv, wrec, brec, eps,
               x_out, rec_out, mu_out, lv_out, z_out):
    def dot(a, b):
        return jnp.dot(a, b, preferred_element_type=jnp.float32)

    st = jnp.concatenate([dot(h0t[...], wx2[...]) + bx2[...],
                          dot(h1mt[...], wn2[...]) + bn2[...]], axis=1)
    sx = jnp.concatenate([dot(h0x[...], wx2[...]) + bx2[...],
                          dot(h1mx[...], wn2[...]) + bn2[...]], axis=1)
    x_out[...] = sx
    mu = dot(st, wmu[...]) + bmu[...]
    lv = dot(st, wlv[...]) + blv[...]
    mu_out[...] = mu
    lv_out[...] = lv
    std = jnp.exp(0.5 * lv)
    z = mu + eps[...] * std
    z_out[...] = z
    r = dot(z, wrec[...]) + brec[...]
    rec_out[...] = 1.0 / (1.0 + jnp.exp(-r))


_post = pl.pallas_call(
    _post_body,
    out_shape=[jax.ShapeDtypeStruct((B, HID), jnp.float32)] * 5,
)


def kernel(ids, adj_tail, adj, feats, W_x1, b_x1, W_n1, b_n1,
           W_x2, b_x2, W_n2, b_n2, W_mu, b_mu, W_lv, b_lv, W_rec, b_rec):
    wcat = jnp.concatenate([W_x1, W_n1], axis=1)
    bcat = jnp.concatenate([b_x1, b_n1]).reshape(1, HID)
    pq, pn = _pre(feats, wcat, bcat)
    cols, eps = _sample_consts()
    rep10 = jnp.asarray(np.arange(B * NS, dtype=np.int32) // NS)
    rep100 = jnp.asarray(np.arange(B * NS * NS, dtype=np.int32) // (NS * NS))

    def expand(s):
        c1, c2 = cols[(s, 0)], cols[(s, 1)]
        cA = jnp.tile(c1, B)
        cB = jnp.tile(jnp.repeat(c1, NS), B)
        cC = jnp.tile(c2, B * NS)
        return cA, cB, cC

    cAt, cBt, cCt = expand(0)
    cAx, cBx, cCx = expand(1)
    h0t, h1mt, h0x, h1mx = _sc(
        ids.astype(jnp.int32), adj_tail.reshape(-1), adj.reshape(-1),
        rep10, rep100, cAt, cBt, cCt, cAx, cBx, cCx, pq, pn)
    x, rec, mu, lv, z = _post(
        h0t, h1mt, h0x, h1mx,
        W_x2, b_x2.reshape(1, H), W_n2, b_n2.reshape(1, H),
        W_mu, b_mu.reshape(1, HID), W_lv, b_lv.reshape(1, HID),
        W_rec, b_rec.reshape(1, HID), eps)
    return (x, rec, mu, lv, z)


# speedup vs baseline: 7.1330x; 1.1483x over previous
import functools

import jax
import jax.numpy as jnp
import numpy as np
from jax import lax
from jax.experimental import pallas as pl
from jax.experimental.pallas import tpu as pltpu
from jax.experimental.pallas import tpu_sc as plsc

N = 10000
D = 256
DEG = 16
B = 1024
NS = 10
H = 128
HID = 2 * H

NW = 32
RB = B // NW
L1 = RB * NS
L2 = L1 * NS


def _sample_consts():
    cpu = jax.local_devices(backend="cpu")[0]
    with jax.default_device(cpu):
        key = jax.random.key(42)
        cols = {}
        for s in range(2):
            ks = jax.random.fold_in(key, s)
            for i in range(2):
                k = jax.random.fold_in(ks, i)
                cols[(s, i)] = np.asarray(
                    jax.random.permutation(k, DEG)[:NS], dtype=np.int32)
        eps = np.asarray(
            jax.random.normal(jax.random.fold_in(key, 2), (B, HID),
                              jnp.float32))
    return cols, eps


_COLS, _EPS = _sample_consts()
_REP10 = (np.arange(B * NS, dtype=np.int32) // NS)
_REP100 = (np.arange(B * NS * NS, dtype=np.int32) // (NS * NS))


def _expand(s):
    c1, c2 = _COLS[(s, 0)], _COLS[(s, 1)]
    cA = np.tile(c1, B)
    cB = np.tile(np.repeat(c1, NS), B)
    cC = np.tile(c2, B * NS)
    return cA, cB, cC


_CAT, _CBT, _CCT = _expand(0)
_CAX, _CBX, _CCX = _expand(1)


def _pre_body(feats_ref, w_ref, b_ref, pq_ref, p_ref):
    acc = jnp.dot(feats_ref[...], w_ref[...],
                  preferred_element_type=jnp.float32) + b_ref[...]
    pq_ref[...] = acc
    p_ref[...] = acc[:, H:]


_pre = pl.pallas_call(
    _pre_body,
    grid=(10,),
    in_specs=[pl.BlockSpec((N // 10, D), lambda i: (i, 0)),
              pl.BlockSpec((D, HID), lambda i: (0, 0)),
              pl.BlockSpec((1, HID), lambda i: (0, 0))],
    out_specs=[pl.BlockSpec((N // 10, HID), lambda i: (i, 0)),
               pl.BlockSpec((N // 10, H), lambda i: (i, 0))],
    out_shape=[jax.ShapeDtypeStruct((N, HID), jnp.float32),
               jax.ShapeDtypeStruct((N, H), jnp.float32)],
)


def _sc_body(ids_hbm, adjt_hbm, adjx_hbm, r10_hbm, r100_hbm,
             cAt_hbm, cBt_hbm, cCt_hbm, cAx_hbm, cBx_hbm, cCx_hbm,
             pq_hbm, pn_hbm,
             h0t_hbm, h1mt_hbm, h0x_hbm, h1mx_hbm,
             ids_v, r10_v, r100_v, idr10_v, idr100_v,
             cAt_v, cAx_v, cB_v, cC_v, f1t_v, f1x_v, frt_v, frx_v,
             cur1r_v, cur1t_v, cur1x_v, cur2t_v, cur2x_v,
             pq0, pq1, gbuf0, gbuf1, h0_acc, h1m_acc,
             sA, sB, sC, sD, sE, sF, sG, sH, sP, sg0, sg1):
    wid = lax.axis_index("s") * 2 + lax.axis_index("c")
    base = wid * RB
    tenth = jnp.float32(0.1)

    def _fcomp(dst, srcA, srcB, nslices):
        def body(t, _):
            sl = pl.ds(16 * t, 16)
            dst[sl] = (srcA[sl] << 4) + srcB[sl]
            return 0
        lax.fori_loop(0, nslices, body, 0)

    a_ids = pltpu.async_copy(ids_hbm.at[pl.ds(base, RB)], ids_v, sA)
    a_r10 = pltpu.async_copy(r10_hbm.at[pl.ds(base * NS, L1)], r10_v, sB)
    a_r100 = pltpu.async_copy(r100_hbm.at[pl.ds(base * NS * NS, L2)],
                              r100_v, sC)
    a_cAt = pltpu.async_copy(cAt_hbm.at[pl.ds(base * NS, L1)], cAt_v, sD)
    a_cAx = pltpu.async_copy(cAx_hbm.at[pl.ds(base * NS, L1)], cAx_v, sE)
    a_cB = pltpu.async_copy(cBt_hbm.at[pl.ds(base * NS * NS, L2)], cB_v, sF)
    a_cC = pltpu.async_copy(cCt_hbm.at[pl.ds(base * NS * NS, L2)], cC_v, sG)
    a_ids.wait()
    a_pq0 = pltpu.async_copy(pq_hbm.at[ids_v], pq0, sH)
    a_r10.wait()
    a_idr10 = pltpu.async_copy(ids_hbm.at[r10_v], idr10_v, sA)
    a_r100.wait()
    a_idr100 = pltpu.async_copy(ids_hbm.at[r100_v], idr100_v, sB)
    a_idr10.wait()
    a_cAt.wait()
    _fcomp(f1t_v, idr10_v, cAt_v, L1 // 16)
    a_cur1t = pltpu.async_copy(adjt_hbm.at[f1t_v], cur1t_v, sA)
    a_cAx.wait()
    _fcomp(f1x_v, idr10_v, cAx_v, L1 // 16)
    a_cur1x = pltpu.async_copy(adjx_hbm.at[f1x_v], cur1x_v, sE)
    a_idr100.wait()
    a_cB.wait()
    _fcomp(frt_v, idr100_v, cB_v, L2 // 16)
    a_c1rt = pltpu.async_copy(adjt_hbm.at[frt_v], cur1r_v, sB)
    a_cBx = pltpu.async_copy(cBx_hbm.at[pl.ds(base * NS * NS, L2)], cB_v, sF)
    a_c1rt.wait()
    a_cC.wait()
    _fcomp(frt_v, cur1r_v, cC_v, L2 // 16)
    a_cur2t = pltpu.async_copy(adjt_hbm.at[frt_v], cur2t_v, sB)
    a_cCx = pltpu.async_copy(cCx_hbm.at[pl.ds(base * NS * NS, L2)], cC_v, sG)
    a_cBx.wait()
    _fcomp(frx_v, idr100_v, cB_v, L2 // 16)
    a_c1rx = pltpu.async_copy(adjx_hbm.at[frx_v], cur1r_v, sC)
    a_c1rx.wait()
    a_cCx.wait()
    _fcomp(frx_v, cur1r_v, cC_v, L2 // 16)
    a_cur2x = pltpu.async_copy(adjx_hbm.at[frx_v], cur2x_v, sC)

    gbufs = (gbuf0, gbuf1)
    sgs = (sg0, sg1)

    for bi, (cur1_v, cur2_v, a_cur1, a_cur2, h0_hbm, h1m_hbm) in enumerate((
            (cur1t_v, cur2t_v, a_cur1t, a_cur2t, h0t_hbm, h1mt_hbm),
            (cur1x_v, cur2x_v, a_cur1x, a_cur2x, h0x_hbm, h1mx_hbm))):
        a_cur1.wait()
        a_cur2.wait()
        pltpu.async_copy(pn_hbm.at[cur2_v.at[pl.ds(0, 200)]], gbuf0, sg0)
        pltpu.async_copy(pn_hbm.at[cur2_v.at[pl.ds(200, 200)]], gbuf1, sg1)

        def oo_body(oo, _):
            for b in range(2):
                c = 2 * oo + b

                if b == 0:
                    @pl.when((c & 3) == 0)
                    def _reload():
                        pltpu.async_copy(
                            pq_hbm.at[cur1_v.at[pl.ds(80 * (oo >> 1), 80)]],
                            pq1, sP).wait()

                pltpu.make_async_copy(
                    pn_hbm.at[cur2_v.at[pl.ds(0, 200)]], gbufs[b],
                    sgs[b]).wait()

                for q in range(2):
                    g0 = 2 * c + q
                    rowb = (g0 & 7) * NS
                    gb = 100 * q
                    gbuf = gbufs[b]

                    def u_body(u, _, g0=g0, rowb=rowb, gb=gb, gbuf=gbuf):
                        sl = pl.ds(16 * u, 16)
                        slh = pl.ds(128 + 16 * u, 16)
                        xacc = jnp.maximum(pq1[rowb, sl], 0.0)
                        nacc0 = pq1[rowb, slh]
                        sv = gbuf[gb, sl]
                        for rr in range(1, NS):
                            sv = sv + gbuf[gb + rr, sl]
                        nacc = jnp.maximum(sv * tenth, 0.0)
                        for jl in range(1, NS):
                            row = rowb + jl
                            xacc = xacc + jnp.maximum(pq1[row, sl], 0.0)
                            nacc0 = nacc0 + pq1[row, slh]
                            sv = gbuf[gb + NS * jl, sl]
                            for rr in range(1, NS):
                                sv = sv + gbuf[gb + NS * jl + rr, sl]
                            nacc = nacc + jnp.maximum(sv * tenth, 0.0)
                        h1m_acc[g0, sl] = xacc * tenth
                        h1m_acc[g0, slh] = nacc * tenth
                        h0_acc[g0, slh] = nacc0 * tenth
                        return 0
                    lax.fori_loop(0, 8, u_body, 0)

                @pl.when(oo < 7)
                def _next():
                    pltpu.async_copy(
                        pn_hbm.at[cur2_v.at[pl.ds(200 * (c + 2), 200)]],
                        gbufs[b], sgs[b])
            return 0
        lax.fori_loop(0, 8, oo_body, 0)

        if bi == 0:
            a_pq0.wait()

        def fin(i, _):
            for u in range(8):
                sl = pl.ds(16 * u, 16)
                slh = pl.ds(128 + 16 * u, 16)
                h0_acc[i, sl] = jnp.maximum(pq0[i, sl], 0.0)
                h0_acc[i, slh] = jnp.maximum(h0_acc[i, slh], 0.0)
            return 0
        lax.fori_loop(0, RB, fin, 0)

        pltpu.sync_copy(h0_acc, h0_hbm.at[pl.ds(base, RB)])
        pltpu.sync_copy(h1m_acc, h1m_hbm.at[pl.ds(base, RB)])


_sc = functools.partial(
    pl.kernel,
    mesh=plsc.VectorSubcoreMesh(core_axis_name="c", subcore_axis_name="s"),
    out_type=[jax.ShapeDtypeStruct((B, HID), jnp.float32)] * 4,
    scratch_types=[
        pltpu.VMEM((RB,), jnp.int32),
        pltpu.VMEM((L1,), jnp.int32),
        pltpu.VMEM((L2,), jnp.int32),
        pltpu.VMEM((L1,), jnp.int32),
        pltpu.VMEM((L2,), jnp.int32),
        pltpu.VMEM((L1,), jnp.int32),
        pltpu.VMEM((L1,), jnp.int32),
        pltpu.VMEM((L2,), jnp.int32),
        pltpu.VMEM((L2,), jnp.int32),
        pltpu.VMEM((L1,), jnp.int32),
        pltpu.VMEM((L1,), jnp.int32),
        pltpu.VMEM((L2,), jnp.int32),
        pltpu.VMEM((L2,), jnp.int32),
        pltpu.VMEM((L2,), jnp.int32),
        pltpu.VMEM((L1,), jnp.int32),
        pltpu.VMEM((L1,), jnp.int32),
        pltpu.VMEM((L2,), jnp.int32),
        pltpu.VMEM((L2,), jnp.int32),
        pltpu.VMEM((RB, HID), jnp.float32),
        pltpu.VMEM((80, HID), jnp.float32),
        pltpu.VMEM((200, H), jnp.float32),
        pltpu.VMEM((200, H), jnp.float32),
        pltpu.VMEM((RB, HID), jnp.float32),
        pltpu.VMEM((RB, HID), jnp.float32),
    ] + [pltpu.SemaphoreType.DMA] * 11,
)(_sc_body)


def _post_body(h0t, h1mt, h0x, h1mx, wx2, bx2, wn2, bn2,
               wmu, bmu, wlv, blv, wrec, brec, eps,
               x_out, rec_out, mu_out, lv_out, z_out):
    def dot(a, b):
        return jnp.dot(a, b, preferred_element_type=jnp.float32)

    st = jnp.concatenate([dot(h0t[...], wx2[...]) + bx2[...],
                          dot(h1mt[...], wn2[...]) + bn2[...]], axis=1)
    sx = jnp.concatenate([dot(h0x[...], wx2[...]) + bx2[...],
                          dot(h1mx[...], wn2[...]) + bn2[...]], axis=1)
    x_out[...] = sx
    mu = dot(st, wmu[...]) + bmu[...]
    lv = dot(st, wlv[...]) + blv[...]
    mu_out[...] = mu
    lv_out[...] = lv
    std = jnp.exp(0.5 * lv)
    z = mu + eps[...] * std
    z_out[...] = z
    r = dot(z, wrec[...]) + brec[...]
    rec_out[...] = 1.0 / (1.0 + jnp.exp(-r))


_post = pl.pallas_call(
    _post_body,
    out_shape=[jax.ShapeDtypeStruct((B, HID), jnp.float32)] * 5,
)


def kernel(ids, adj_tail, adj, feats, W_x1, b_x1, W_n1, b_n1,
           W_x2, b_x2, W_n2, b_n2, W_mu, b_mu, W_lv, b_lv, W_rec, b_rec):
    wcat = jnp.concatenate([W_x1, W_n1], axis=1)
    bcat = jnp.concatenate([b_x1, b_n1]).reshape(1, HID)
    pq, pn = _pre(feats, wcat, bcat)
    h0t, h1mt, h0x, h1mx = _sc(
        ids.astype(jnp.int32), adj_tail.reshape(-1), adj.reshape(-1),
        _REP10, _REP100, _CAT, _CBT, _CCT, _CAX, _CBX, _CCX, pq, pn)
    x, rec, mu, lv, z = _post(
        h0t, h1mt, h0x, h1mx,
        W_x2, b_x2.reshape(1, H), W_n2, b_n2.reshape(1, H),
        W_mu, b_mu.reshape(1, HID), W_lv, b_lv.reshape(1, HID),
        W_rec, b_rec.reshape(1, HID), _EPS)
    return (x, rec, mu, lv, z)


# speedup vs baseline: 7.2711x; 1.0194x over previous
import functools

import jax
import jax.numpy as jnp
import numpy as np
from jax import lax
from jax.experimental import pallas as pl
from jax.experimental.pallas import tpu as pltpu
from jax.experimental.pallas import tpu_sc as plsc

N = 10000
D = 256
DEG = 16
B = 1024
NS = 10
H = 128
HID = 2 * H

NW = 32
RB = B // NW
L1 = RB * NS
L2 = L1 * NS


_COLS = {
    (0, 0): np.array([3, 2, 10, 14, 6, 15, 1, 0, 13, 5], np.int32),
    (0, 1): np.array([1, 8, 9, 2, 0, 10, 13, 11, 6, 15], np.int32),
    (1, 0): np.array([5, 13, 9, 11, 3, 15, 4, 1, 8, 10], np.int32),
    (1, 1): np.array([2, 11, 14, 15, 6, 1, 12, 13, 7, 8], np.int32),
}
_REP10 = (np.arange(B * NS, dtype=np.int32) // NS)
_REP100 = (np.arange(B * NS * NS, dtype=np.int32) // (NS * NS))


def _expand(s):
    c1, c2 = _COLS[(s, 0)], _COLS[(s, 1)]
    cA = np.tile(c1, B)
    cB = np.tile(np.repeat(c1, NS), B)
    cC = np.tile(c2, B * NS)
    return cA, cB, cC


_CAT, _CBT, _CCT = _expand(0)
_CAX, _CBX, _CCX = _expand(1)


def _pre_body(feats_ref, w_ref, b_ref, pq_ref, p_ref):
    acc = jnp.dot(feats_ref[...], w_ref[...],
                  preferred_element_type=jnp.float32) + b_ref[...]
    pq_ref[...] = acc
    p_ref[...] = acc[:, H:]


_pre = pl.pallas_call(
    _pre_body,
    grid=(10,),
    in_specs=[pl.BlockSpec((N // 10, D), lambda i: (i, 0)),
              pl.BlockSpec((D, HID), lambda i: (0, 0)),
              pl.BlockSpec((1, HID), lambda i: (0, 0))],
    out_specs=[pl.BlockSpec((N // 10, HID), lambda i: (i, 0)),
               pl.BlockSpec((N // 10, H), lambda i: (i, 0))],
    out_shape=[jax.ShapeDtypeStruct((N, HID), jnp.float32),
               jax.ShapeDtypeStruct((N, H), jnp.float32)],
)


def _sc_body(ids_hbm, adjt_hbm, adjx_hbm, r10_hbm, r100_hbm,
             cAt_hbm, cBt_hbm, cCt_hbm, cAx_hbm, cBx_hbm, cCx_hbm,
             pq_hbm, pn_hbm,
             h0t_hbm, h1mt_hbm, h0x_hbm, h1mx_hbm,
             ids_v, r10_v, r100_v, idr10_v, idr100_v,
             cAt_v, cAx_v, cB_v, cC_v, f1t_v, f1x_v, frt_v, frx_v,
             cur1r_v, cur1t_v, cur1x_v, cur2t_v, cur2x_v,
             pq0, pq1a, pq1b, gbuf0, gbuf1, h0_acc, h1m_acc,
             sA, sB, sC, sD, sE, sF, sG, sH, sP, sg0, sg1, sN):
    sid = lax.axis_index("s")
    wid = sid * 2 + lax.axis_index("c")
    base = wid * RB
    tenth = jnp.float32(0.1)


    def _fcomp(dst, srcA, srcB, nslices):
        def body(t, _):
            sl = pl.ds(16 * t, 16)
            dst[sl] = (srcA[sl] << 4) + srcB[sl]
            return 0
        lax.fori_loop(0, nslices, body, 0)

    a_ids = pltpu.async_copy(ids_hbm.at[pl.ds(base, RB)], ids_v, sA)
    a_r10 = pltpu.async_copy(r10_hbm.at[pl.ds(base * NS, L1)], r10_v, sB)
    a_r100 = pltpu.async_copy(r100_hbm.at[pl.ds(base * NS * NS, L2)],
                              r100_v, sC)
    a_cAt = pltpu.async_copy(cAt_hbm.at[pl.ds(base * NS, L1)], cAt_v, sD)
    a_cAx = pltpu.async_copy(cAx_hbm.at[pl.ds(base * NS, L1)], cAx_v, sE)
    a_cB = pltpu.async_copy(cBt_hbm.at[pl.ds(base * NS * NS, L2)], cB_v, sF)
    a_cC = pltpu.async_copy(cCt_hbm.at[pl.ds(base * NS * NS, L2)], cC_v, sG)
    a_ids.wait()
    a_pq0 = pltpu.async_copy(pq_hbm.at[ids_v], pq0, sH)
    a_r10.wait()
    a_idr10 = pltpu.async_copy(ids_hbm.at[r10_v], idr10_v, sA)
    a_r100.wait()
    a_idr100 = pltpu.async_copy(ids_hbm.at[r100_v], idr100_v, sB)
    a_idr10.wait()
    a_cAt.wait()
    _fcomp(f1t_v, idr10_v, cAt_v, L1 // 16)
    a_cur1t = pltpu.async_copy(adjt_hbm.at[f1t_v], cur1t_v, sA)
    a_cAx.wait()
    _fcomp(f1x_v, idr10_v, cAx_v, L1 // 16)
    a_cur1x = pltpu.async_copy(adjx_hbm.at[f1x_v], cur1x_v, sE)
    a_idr100.wait()
    a_cB.wait()
    _fcomp(frt_v, idr100_v, cB_v, L2 // 16)
    a_c1rt = pltpu.async_copy(adjt_hbm.at[frt_v], cur1r_v, sB)
    a_cBx = pltpu.async_copy(cBx_hbm.at[pl.ds(base * NS * NS, L2)], cB_v, sF)
    a_c1rt.wait()
    a_cC.wait()
    _fcomp(frt_v, cur1r_v, cC_v, L2 // 16)
    a_cur2t = pltpu.async_copy(adjt_hbm.at[frt_v], cur2t_v, sB)
    a_cCx = pltpu.async_copy(cCx_hbm.at[pl.ds(base * NS * NS, L2)], cC_v, sG)
    a_cBx.wait()
    _fcomp(frx_v, idr100_v, cB_v, L2 // 16)
    a_c1rx = pltpu.async_copy(adjx_hbm.at[frx_v], cur1r_v, sC)
    a_c1rx.wait()
    a_cCx.wait()
    _fcomp(frx_v, cur1r_v, cC_v, L2 // 16)
    a_cur2x = pltpu.async_copy(adjx_hbm.at[frx_v], cur2x_v, sC)

    gbufs = (gbuf0, gbuf1)
    sgs = (sg0, sg1)
    pq1s = (pq1a, pq1b)
    sps = (sP, sN)

    for bi, (cur1_v, cur2_v, a_cur1, a_cur2, h0_hbm, h1m_hbm) in enumerate((
            (cur1t_v, cur2t_v, a_cur1t, a_cur2t, h0t_hbm, h1mt_hbm),
            (cur1x_v, cur2x_v, a_cur1x, a_cur2x, h0x_hbm, h1mx_hbm))):
        a_cur1.wait()
        a_cur2.wait()
        pltpu.async_copy(pn_hbm.at[cur2_v.at[pl.ds(0, 200)]], gbuf0, sg0)
        pltpu.async_copy(pn_hbm.at[cur2_v.at[pl.ds(200, 200)]], gbuf1, sg1)
        pltpu.async_copy(pq_hbm.at[cur1_v.at[pl.ds(0, 40)]], pq1a, sP)
        pltpu.async_copy(pq_hbm.at[cur1_v.at[pl.ds(40, 40)]], pq1b, sN)

        def oo2_body(oo2, _):
            for p2 in range(2):
                oo = 2 * oo2 + p2
                pq1 = pq1s[p2]
                pltpu.make_async_copy(
                    pq_hbm.at[cur1_v.at[pl.ds(0, 40)]], pq1,
                    sps[p2]).wait()
                for b in range(2):
                    c = 2 * oo + b

                    pltpu.make_async_copy(
                        pn_hbm.at[cur2_v.at[pl.ds(0, 200)]], gbufs[b],
                        sgs[b]).wait()

                    for q in range(2):
                        g0 = 2 * c + q
                        rowb = (g0 & 3) * NS
                        gb = 100 * q
                        gbuf = gbufs[b]

                        def u_body(u, _, g0=g0, rowb=rowb, gb=gb,
                                   gbuf=gbuf, pq1=pq1):
                            sl = pl.ds(16 * u, 16)
                            slh = pl.ds(128 + 16 * u, 16)
                            xacc = jnp.maximum(pq1[rowb, sl], 0.0)
                            nacc0 = pq1[rowb, slh]
                            sv = gbuf[gb, sl]
                            for rr in range(1, NS):
                                sv = sv + gbuf[gb + rr, sl]
                            nacc = jnp.maximum(sv * tenth, 0.0)
                            for jl in range(1, NS):
                                row = rowb + jl
                                xacc = xacc + jnp.maximum(pq1[row, sl], 0.0)
                                nacc0 = nacc0 + pq1[row, slh]
                                sv = gbuf[gb + NS * jl, sl]
                                for rr in range(1, NS):
                                    sv = sv + gbuf[gb + NS * jl + rr, sl]
                                nacc = nacc + jnp.maximum(sv * tenth, 0.0)
                            h1m_acc[g0, sl] = xacc * tenth
                            h1m_acc[g0, slh] = nacc * tenth
                            h0_acc[g0, slh] = nacc0 * tenth
                            return 0
                        lax.fori_loop(0, 8, u_body, 0)

                    @pl.when(c < 14)
                    def _next():
                        pltpu.async_copy(
                            pn_hbm.at[cur2_v.at[pl.ds(200 * (c + 2), 200)]],
                            gbufs[b], sgs[b])

                @pl.when(oo2 < 3)
                def _next_pq():
                    pltpu.async_copy(
                        pq_hbm.at[cur1_v.at[pl.ds(40 * (oo + 2), 40)]],
                        pq1, sps[p2])
            return 0
        lax.fori_loop(0, 4, oo2_body, 0)

        if bi == 0:
            a_pq0.wait()

        def fin(i, _):
            for u in range(8):
                sl = pl.ds(16 * u, 16)
                slh = pl.ds(128 + 16 * u, 16)
                h0_acc[i, sl] = jnp.maximum(pq0[i, sl], 0.0)
                h0_acc[i, slh] = jnp.maximum(h0_acc[i, slh], 0.0)
            return 0
        lax.fori_loop(0, RB, fin, 0)

        pltpu.sync_copy(h0_acc, h0_hbm.at[pl.ds(base, RB)])
        pltpu.sync_copy(h1m_acc, h1m_hbm.at[pl.ds(base, RB)])


_sc = functools.partial(
    pl.kernel,
    mesh=plsc.VectorSubcoreMesh(core_axis_name="c", subcore_axis_name="s"),
    out_type=[jax.ShapeDtypeStruct((B, HID), jnp.float32)] * 4,
    scratch_types=[
        pltpu.VMEM((RB,), jnp.int32),
        pltpu.VMEM((L1,), jnp.int32),
        pltpu.VMEM((L2,), jnp.int32),
        pltpu.VMEM((L1,), jnp.int32),
        pltpu.VMEM((L2,), jnp.int32),
        pltpu.VMEM((L1,), jnp.int32),
        pltpu.VMEM((L1,), jnp.int32),
        pltpu.VMEM((L2,), jnp.int32),
        pltpu.VMEM((L2,), jnp.int32),
        pltpu.VMEM((L1,), jnp.int32),
        pltpu.VMEM((L1,), jnp.int32),
        pltpu.VMEM((L2,), jnp.int32),
        pltpu.VMEM((L2,), jnp.int32),
        pltpu.VMEM((L2,), jnp.int32),
        pltpu.VMEM((L1,), jnp.int32),
        pltpu.VMEM((L1,), jnp.int32),
        pltpu.VMEM((L2,), jnp.int32),
        pltpu.VMEM((L2,), jnp.int32),
        pltpu.VMEM((RB, HID), jnp.float32),
        pltpu.VMEM((40, HID), jnp.float32),
        pltpu.VMEM((40, HID), jnp.float32),
        pltpu.VMEM((200, H), jnp.float32),
        pltpu.VMEM((200, H), jnp.float32),
        pltpu.VMEM((RB, HID), jnp.float32),
        pltpu.VMEM((RB, HID), jnp.float32),
    ] + [pltpu.SemaphoreType.DMA] * 12,
)(_sc_body)


def _post_body(h0t, h1mt, h0x, h1mx, wx2, bx2, wn2, bn2,
               wmu, bmu, wlv, blv, wrec, brec, eps,
               x_out, rec_out, mu_out, lv_out, z_out):
    def dot(a, b):
        return jnp.dot(a, b, preferred_element_type=jnp.float32)

    st = jnp.concatenate([dot(h0t[...], wx2[...]) + bx2[...],
                          dot(h1mt[...], wn2[...]) + bn2[...]], axis=1)
    sx = jnp.concatenate([dot(h0x[...], wx2[...]) + bx2[...],
                          dot(h1mx[...], wn2[...]) + bn2[...]], axis=1)
    x_out[...] = sx
    mu = dot(st, wmu[...]) + bmu[...]
    lv = dot(st, wlv[...]) + blv[...]
    mu_out[...] = mu
    lv_out[...] = lv
    std = jnp.exp(0.5 * lv)
    z = mu + eps[...] * std
    z_out[...] = z
    r = dot(z, wrec[...]) + brec[...]
    rec_out[...] = 1.0 / (1.0 + jnp.exp(-r))


_post = pl.pallas_call(
    _post_body,
    out_shape=[jax.ShapeDtypeStruct((B, HID), jnp.float32)] * 5,
)


def kernel(ids, adj_tail, adj, feats, W_x1, b_x1, W_n1, b_n1,
           W_x2, b_x2, W_n2, b_n2, W_mu, b_mu, W_lv, b_lv, W_rec, b_rec):
    wcat = jnp.concatenate([W_x1, W_n1], axis=1)
    bcat = jnp.concatenate([b_x1, b_n1]).reshape(1, HID)
    pq, pn = _pre(feats, wcat, bcat)
    eps = jax.random.normal(jax.random.fold_in(jax.random.key(42), 2),
                            (B, HID), jnp.float32)
    h0t, h1mt, h0x, h1mx = _sc(
        ids.astype(jnp.int32), adj_tail.reshape(-1), adj.reshape(-1),
        _REP10, _REP100, _CAT, _CBT, _CCT, _CAX, _CBX, _CCX, pq, pn)
    x, rec, mu, lv, z = _post(
        h0t, h1mt, h0x, h1mx,
        W_x2, b_x2.reshape(1, H), W_n2, b_n2.reshape(1, H),
        W_mu, b_mu.reshape(1, HID), W_lv, b_lv.reshape(1, HID),
        W_rec, b_rec.reshape(1, HID), eps)
    return (x, rec, mu, lv, z)
